# baseline (speedup 1.0000x reference)
import functools

import jax
import jax.numpy as jnp
from jax import lax
from jax.experimental import pallas as pl
from jax.experimental.pallas import tpu as pltpu
from jax.experimental.pallas import tpu_sc as plsc

NNODE = 10000
NEDGE = 320000
DIN = 128
HID = 64
NOUT = 2

NC = 2
NS = 16
L = 16
NW = NC * NS
EPT = NEDGE // NW
C1 = 80
NCH = EPT // C1

_f32 = jnp.float32

_mesh = plsc.VectorSubcoreMesh(core_axis_name="c", subcore_axis_name="s")
_sc_params = pltpu.CompilerParams(needs_layout_passes=False,
                                  use_tc_tiling_on_sc=False)


def _leaky(x):
    return jnp.where(x >= 0.0, x, 0.2 * x)


def _dense1_body(x_ref, w_ref, asv_ref, adv_ref, h_ref, as_ref, ad_ref):
    h = jnp.dot(x_ref[...], w_ref[...], preferred_element_type=_f32)
    h_ref[...] = h
    as_ref[...] = jnp.sum(h * asv_ref[...], axis=1)
    ad_ref[...] = jnp.sum(h * adv_ref[...], axis=1)


def _dense1(x, W1, asv, adv):
    return pl.pallas_call(
        _dense1_body,
        out_shape=[
            jax.ShapeDtypeStruct((NNODE, HID), _f32),
            jax.ShapeDtypeStruct((NNODE,), _f32),
            jax.ShapeDtypeStruct((NNODE,), _f32),
        ],
    )(x, W1, asv, adv)


NBUF = 4


def _edge1_body(h_hbm, as_hbm, ad_hbm, ei_hbm, zro_hbm,
                acc_hbm, den_hbm,
                as_v, ad_v, den_v, srcall_v, dstall_v,
                rows0_v, rows1_v, rows2_v, rows3_v,
                dstb0_v, dstb1_v, dstb2_v, dstb3_v, acc_sh,
                gsem0, gsem1, gsem2, gsem3,
                ssem0, ssem1, ssem2, ssem3):
    cid = lax.axis_index("c")
    sid = lax.axis_index("s")
    wid = cid * NS + sid

    rows = (rows0_v, rows1_v, rows2_v, rows3_v)
    dstb = (dstb0_v, dstb1_v, dstb2_v, dstb3_v)
    gsem = (gsem0, gsem1, gsem2, gsem3)
    ssem = (ssem0, ssem1, ssem2, ssem3)

    pltpu.sync_copy(as_hbm, as_v)
    pltpu.sync_copy(ad_hbm, ad_v)
    pltpu.sync_copy(ei_hbm.at[pl.ds(wid * EPT, EPT)], srcall_v)
    pltpu.sync_copy(ei_hbm.at[pl.ds(NEDGE + wid * EPT, EPT)], dstall_v)

    def _z(i, carry):
        den_v[pl.ds(i * L, L)] = jnp.zeros((L,), _f32)
        return carry

    lax.fori_loop(0, NNODE // L, _z, 0)

    @pl.when(sid == 0)
    def _():
        pltpu.sync_copy(zro_hbm, acc_sh)

    plsc.subcore_barrier()

    def _gather(k, p):
        pltpu.async_copy(h_hbm.at[srcall_v.at[pl.ds(k * C1, C1)]],
                         rows[p], gsem[p])

    def _wait_gather(p):
        pltpu.make_async_copy(h_hbm.at[srcall_v.at[pl.ds(0, C1)]],
                              rows[p], gsem[p]).wait()

    def _scatter(p):
        pltpu.async_copy(rows[p], acc_sh.at[dstb[p]], ssem[p], add=True)

    def _wait_scatter(p):
        pltpu.make_async_copy(rows[p], acc_sh.at[dstb[p]], ssem[p]).wait()

    def _compute(k, p):
        rv = rows[p]
        db = dstb[p]
        for g in range(C1 // L):
            s16 = srcall_v[pl.ds(k * C1 + g * L, L)]
            d16 = dstall_v[pl.ds(k * C1 + g * L, L)]
            db[pl.ds(g * L, L)] = d16
            w = jnp.exp(_leaky(plsc.load_gather(as_v, [s16]) +
                               plsc.load_gather(ad_v, [d16])))
            plsc.addupdate_scatter(den_v, [d16], w)
            for j in range(L):
                ej = g * L + j
                wb = jnp.zeros((L,), _f32) + w[j]
                for c in range(HID // L):
                    sl = pl.ds(c * L, L)
                    rv[ej, sl] = rv[ej, sl] * wb

    for p in range(NBUF):
        _gather(p, p)
    _wait_gather(0)
    _compute(0, 0)
    _scatter(0)

    def _quad(i, carry):
        a = 4 * i + 1
        for q in range(NBUF):
            p = (1 + q) % NBUF
            pprev = (p + 3) % NBUF
            _wait_scatter(pprev)

            @pl.when(a + q + 3 < NCH)
            def _():
                _gather(a + q + 3, pprev)

            _wait_gather(p)
            _compute(a + q, p)
            _scatter(p)
        return carry

    lax.fori_loop(0, (NCH - 1) // NBUF, _quad, 0)

    _wait_scatter((NCH - 1) % NBUF)

    plsc.subcore_barrier()

    @pl.when(sid == 0)
    def _():
        pltpu.sync_copy(acc_sh, acc_hbm.at[cid])

    pltpu.sync_copy(den_v, den_hbm.at[pl.ds(wid * NNODE, NNODE)])


@functools.partial(
    pl.kernel,
    out_type=[
        jax.ShapeDtypeStruct((NC, NNODE, HID), _f32),
        jax.ShapeDtypeStruct((NW * NNODE,), _f32),
    ],
    mesh=_mesh,
    compiler_params=_sc_params,
    scratch_types=[
        pltpu.VMEM((NNODE,), _f32),
        pltpu.VMEM((NNODE,), _f32),
        pltpu.VMEM((NNODE,), _f32),
        pltpu.VMEM((EPT,), jnp.int32),
        pltpu.VMEM((EPT,), jnp.int32),
        pltpu.VMEM((C1, HID), _f32),
        pltpu.VMEM((C1, HID), _f32),
        pltpu.VMEM((C1, HID), _f32),
        pltpu.VMEM((C1, HID), _f32),
        pltpu.VMEM((C1,), jnp.int32),
        pltpu.VMEM((C1,), jnp.int32),
        pltpu.VMEM((C1,), jnp.int32),
        pltpu.VMEM((C1,), jnp.int32),
        pltpu.VMEM_SHARED((NNODE, HID), _f32),
        pltpu.SemaphoreType.DMA,
        pltpu.SemaphoreType.DMA,
        pltpu.SemaphoreType.DMA,
        pltpu.SemaphoreType.DMA,
        pltpu.SemaphoreType.DMA,
        pltpu.SemaphoreType.DMA,
        pltpu.SemaphoreType.DMA,
        pltpu.SemaphoreType.DMA,
    ],
)
def _edge1(h_hbm, as_hbm, ad_hbm, ei_hbm, zro_hbm,
           acc_hbm, den_hbm, *rest):
    _edge1_body(h_hbm, as_hbm, ad_hbm, ei_hbm, zro_hbm,
                acc_hbm, den_hbm, *rest)


def _mid_body(acc0_ref, acc1_ref, denp_ref, as1_ref, ad1_ref,
              h1_ref, b1_ref, w2t_ref, a2s_ref, a2d_ref,
              h2m0_ref, h2m1_ref, as2_ref, ad2_ref):
    w = jnp.exp(_leaky(as1_ref[...] + ad1_ref[...]))
    den = jnp.sum(denp_ref[...], axis=0) + w + 1e-16
    num = acc0_ref[...] + acc1_ref[...] + w[:, None] * h1_ref[...]
    h2 = jnp.maximum(num / den[:, None] + b1_ref[...], 0.0)
    h2m0 = jnp.sum(h2 * w2t_ref[0:1, :], axis=1)
    h2m1 = jnp.sum(h2 * w2t_ref[1:2, :], axis=1)
    h2m0_ref[...] = h2m0
    h2m1_ref[...] = h2m1
    as2_ref[...] = h2m0 * a2s_ref[0, 0] + h2m1 * a2s_ref[0, 1]
    ad2_ref[...] = h2m0 * a2d_ref[0, 0] + h2m1 * a2d_ref[0, 1]


def _mid(acc0, acc1, denp, as1, ad1, h1, b1, W2t, a2s, a2d):
    return pl.pallas_call(
        _mid_body,
        out_shape=[
            jax.ShapeDtypeStruct((NNODE,), _f32),
            jax.ShapeDtypeStruct((NNODE,), _f32),
            jax.ShapeDtypeStruct((NNODE,), _f32),
            jax.ShapeDtypeStruct((NNODE,), _f32),
        ],
    )(acc0, acc1, denp, as1, ad1, h1, b1, W2t, a2s, a2d)


def _edge2_body(h0_hbm, h1_hbm, as_hbm, ad_hbm, ei_hbm,
                red_hbm,
                h0_v, h1_v, as_v, ad_v, src_v, dst_v, d_v, a0_v, a1_v):
    cid = lax.axis_index("c")
    sid = lax.axis_index("s")
    wid = cid * NS + sid

    pltpu.sync_copy(h0_hbm, h0_v)
    pltpu.sync_copy(h1_hbm, h1_v)
    pltpu.sync_copy(as_hbm, as_v)
    pltpu.sync_copy(ad_hbm, ad_v)
    pltpu.sync_copy(ei_hbm.at[pl.ds(wid * EPT, EPT)], src_v)
    pltpu.sync_copy(ei_hbm.at[pl.ds(NEDGE + wid * EPT, EPT)], dst_v)

    def _z(i, carry):
        z = jnp.zeros((L,), _f32)
        d_v[pl.ds(i * L, L)] = z
        a0_v[pl.ds(i * L, L)] = z
        a1_v[pl.ds(i * L, L)] = z
        return carry

    lax.fori_loop(0, NNODE // L, _z, 0)

    def _grp(g, carry):
        s16 = src_v[pl.ds(g * L, L)]
        d16 = dst_v[pl.ds(g * L, L)]
        w = jnp.exp(_leaky(plsc.load_gather(as_v, [s16]) +
                           plsc.load_gather(ad_v, [d16])))
        plsc.addupdate_scatter(d_v, [d16], w)
        plsc.addupdate_scatter(a0_v, [d16], w * plsc.load_gather(h0_v, [s16]))
        plsc.addupdate_scatter(a1_v, [d16], w * plsc.load_gather(h1_v, [s16]))
        return carry

    lax.fori_loop(0, EPT // L, _grp, 0)

    pltpu.sync_copy(d_v, red_hbm.at[pl.ds((0 * NW + wid) * NNODE, NNODE)])
    pltpu.sync_copy(a0_v, red_hbm.at[pl.ds((1 * NW + wid) * NNODE, NNODE)])
    pltpu.sync_copy(a1_v, red_hbm.at[pl.ds((2 * NW + wid) * NNODE, NNODE)])


@functools.partial(
    pl.kernel,
    out_type=jax.ShapeDtypeStruct((3 * NW * NNODE,), _f32),
    mesh=_mesh,
    compiler_params=_sc_params,
    scratch_types=[
        pltpu.VMEM((NNODE,), _f32),
        pltpu.VMEM((NNODE,), _f32),
        pltpu.VMEM((NNODE,), _f32),
        pltpu.VMEM((NNODE,), _f32),
        pltpu.VMEM((EPT,), jnp.int32),
        pltpu.VMEM((EPT,), jnp.int32),
        pltpu.VMEM((NNODE,), _f32),
        pltpu.VMEM((NNODE,), _f32),
        pltpu.VMEM((NNODE,), _f32),
    ],
)
def _edge2(h0_hbm, h1_hbm, as_hbm, ad_hbm, ei_hbm,
           red_hbm, *rest):
    _edge2_body(h0_hbm, h1_hbm, as_hbm, ad_hbm, ei_hbm,
                red_hbm, *rest)


def _fin_body(pd_ref, p0_ref, p1_ref,
              as2_ref, ad2_ref, h2m0_ref, h2m1_ref, b2_ref, out_ref):
    w = jnp.exp(_leaky(as2_ref[...] + ad2_ref[...]))
    den = jnp.sum(pd_ref[...], axis=0) + w + 1e-16
    o0 = (jnp.sum(p0_ref[...], axis=0) + w * h2m0_ref[...]) / den \
        + b2_ref[0, 0]
    o1 = (jnp.sum(p1_ref[...], axis=0) + w * h2m1_ref[...]) / den \
        + b2_ref[0, 1]
    out_ref[...] = jnp.concatenate(
        [o0[:, None], o1[:, None]], axis=1)


def _fin(pd, p0, p1, as2, ad2, h2m0, h2m1, b2):
    return pl.pallas_call(
        _fin_body,
        out_shape=jax.ShapeDtypeStruct((NNODE, NOUT), _f32),
    )(pd, p0, p1, as2, ad2, h2m0, h2m1, b2)


def kernel(x, edge_index, W1, a1s, a1d, b1, W2, a2s, a2d, b2):
    ei = edge_index.reshape(2 * NEDGE)

    h1, as1, ad1 = _dense1(x, W1, a1s.reshape(1, HID), a1d.reshape(1, HID))

    zro = jnp.zeros((NNODE, HID), _f32)
    acc, denp = _edge1(h1, as1, ad1, ei, zro)

    h2m0, h2m1, as2, ad2 = _mid(acc[0], acc[1],
                                denp.reshape(NW, NNODE),
                                as1, ad1, h1, b1.reshape(1, HID),
                                W2.T, a2s.reshape(1, NOUT),
                                a2d.reshape(1, NOUT))

    red = _edge2(h2m0, h2m1, as2, ad2, ei)
    red = red.reshape(3, NW, NNODE)

    return _fin(red[0], red[1], red[2], as2, ad2, h2m0, h2m1,
                b2.reshape(1, NOUT))
